# baseline (speedup 1.0000x reference)
import jax
import jax.numpy as jnp
from jax.experimental import pallas as pl
from jax.experimental.pallas import tpu as pltpu

N = 512
RCUT = 6.0
RCUT_SMTH = 0.5
SR_MEAN = 0.1
SR_STD = 0.3
NNBRS = 128.0
AXIS = 16
WID1 = 32
WID2 = 64
FIT = 128
OUT_NORM = 1.0
EBIAS = 0.0

BLK = 64
GRP = 8


def _dp_kernel(Ld_ref, Li_ref, cb_ref, cT_ref, W1_ref, b1_ref, W2p_ref,
               b2p_ref, tb_ref, Wf1_ref, bf1_ref, Wf2_ref, bf2_ref, Wf3_ref,
               out_ref):
    i = pl.program_id(0)
    cb = cb_ref[...]
    cT = cT_ref[...]

    diffs = []
    r2 = jnp.full((BLK, N), 1e-16, jnp.float32)
    for d in range(3):
        dd = cb[:, d:d + 1] - cT[d:d + 1, :]
        fr = dd * Li_ref[d]
        fr = fr - jnp.round(fr)
        dd = fr * Ld_ref[d]
        diffs.append(dd)
        r2 = r2 + dd * dd
    r = jnp.sqrt(r2)

    u = (r - RCUT_SMTH) / (RCUT - RCUT_SMTH)
    u = jnp.clip(u, 0.0, 1.0)
    sw = u * u * u * (-6.0 * u * u + 15.0 * u - 10.0) + 1.0
    inv_r = 1.0 / jnp.maximum(r, 1e-8)
    sr = jnp.where(r < RCUT, inv_r * sw, 0.0)
    rows = i * BLK + jax.lax.broadcasted_iota(jnp.int32, (BLK, N), 0)
    cols = jax.lax.broadcasted_iota(jnp.int32, (BLK, N), 1)
    sr = jnp.where(rows == cols, 0.0, sr)

    srn = sr / SR_STD
    snc = (sr - SR_MEAN) / SR_STD
    inv_rr = 1.0 / (r + 1e-16)
    sq3 = 3.0 ** 0.5
    R = [srn] + [sq3 * srn * (diffs[d] * inv_rr) for d in range(3)]

    H = jnp.tanh(W1_ref[...] * snc[None, :, :] + b1_ref[...])

    W2p = W2p_ref[...]
    b2p = b2p_ref[...]

    T = [[] for _ in range(4)]
    for g in range(BLK // GRP):
        Hg = H[:, g * GRP:(g + 1) * GRP, :].reshape(WID1 * GRP, N)
        Gg = jnp.tanh(
            jax.lax.dot_general(W2p, Hg, (((1,), (0,)), ((), ())),
                                preferred_element_type=jnp.float32) + b2p)
        Gg3 = Gg.reshape(GRP, WID2, N)
        for x in range(4):
            Rg = R[x][g * GRP:(g + 1) * GRP, None, :]
            T[x].append(jnp.sum(Gg3 * Rg, axis=2) * (1.0 / NNBRS))
    Tm = [jnp.concatenate(T[x], axis=0) for x in range(4)]

    TN = Tm[0] + tb_ref[...]
    feats = []
    for a in range(AXIS):
        fa = TN * TN[:, a:a + 1]
        for d in range(1, 4):
            fa = fa + Tm[d] * Tm[d][:, a:a + 1]
        feats.append(fa)
    feat = jnp.concatenate(feats, axis=1)

    f1 = jnp.tanh(jnp.dot(feat, Wf1_ref[...],
                          preferred_element_type=jnp.float32) + bf1_ref[...])
    f2 = jnp.tanh(jnp.dot(f1, Wf2_ref[...],
                          preferred_element_type=jnp.float32) + bf2_ref[...])
    v = jnp.dot(jnp.sum(f2, axis=0, keepdims=True), Wf3_ref[...],
                preferred_element_type=jnp.float32)

    @pl.when(i == 0)
    def _init():
        out_ref[...] = jnp.zeros_like(out_ref)

    out_ref[...] += v


def kernel(coord_N3, box_33, W_e1, b_e1, W_e2, b_e2, Tbias,
           W_f1, b_f1, W_f2, b_f2, W_f3, b_f3):
    coordT = coord_N3.T
    Ld = jnp.diagonal(box_33)
    Li = 1.0 / Ld
    W1c = W_e1.reshape(WID1, 1, 1)
    b1c = b_e1.reshape(WID1, 1, 1)
    eye8 = jnp.eye(GRP, dtype=jnp.float32)
    W2p = jnp.einsum('kw,jJ->jwkJ', W_e2, eye8).reshape(GRP * WID2, WID1 * GRP)
    b2p = jnp.tile(b_e2, GRP).reshape(GRP * WID2, 1)
    tb = Tbias.reshape(1, WID2)
    bf1 = b_f1.reshape(1, FIT)
    bf2 = b_f2.reshape(1, FIT)

    grid = (N // BLK,)
    res = pl.pallas_call(
        _dp_kernel,
        grid=grid,
        in_specs=[
            pl.BlockSpec(memory_space=pltpu.SMEM),
            pl.BlockSpec(memory_space=pltpu.SMEM),
            pl.BlockSpec((BLK, 3), lambda i: (i, 0)),
            pl.BlockSpec((3, N), lambda i: (0, 0)),
            pl.BlockSpec((WID1, 1, 1), lambda i: (0, 0, 0)),
            pl.BlockSpec((WID1, 1, 1), lambda i: (0, 0, 0)),
            pl.BlockSpec((GRP * WID2, WID1 * GRP), lambda i: (0, 0)),
            pl.BlockSpec((GRP * WID2, 1), lambda i: (0, 0)),
            pl.BlockSpec((1, WID2), lambda i: (0, 0)),
            pl.BlockSpec((AXIS * WID2, FIT), lambda i: (0, 0)),
            pl.BlockSpec((1, FIT), lambda i: (0, 0)),
            pl.BlockSpec((FIT, FIT), lambda i: (0, 0)),
            pl.BlockSpec((1, FIT), lambda i: (0, 0)),
            pl.BlockSpec((FIT, 1), lambda i: (0, 0)),
        ],
        out_specs=pl.BlockSpec((1, 1), lambda i: (0, 0)),
        out_shape=jax.ShapeDtypeStruct((1, 1), jnp.float32),
    )(Ld, Li, coord_N3, coordT, W1c, b1c, W2p, b2p, tb,
      W_f1, bf1, W_f2, bf2, W_f3)

    return (res[0, 0] + N * (b_f3[0] + EBIAS)) * OUT_NORM


# speedup vs baseline: 1.0346x; 1.0346x over previous
import jax
import jax.numpy as jnp
from jax.experimental import pallas as pl
from jax.experimental.pallas import tpu as pltpu

N = 512
RCUT = 6.0
RCUT_SMTH = 0.5
SR_MEAN = 0.1
SR_STD = 0.3
NNBRS = 128.0
AXIS = 16
WID1 = 32
WID2 = 64
FIT = 128
OUT_NORM = 1.0
EBIAS = 0.0

BLK = 128
GRP = 8


def _dp_kernel(Ld_ref, Li_ref, cb_ref, cT_ref, W1_ref, b1_ref, W2p_ref,
               b2p_ref, tb_ref, Wf1_ref, bf1_ref, Wf2_ref, bf2_ref, Wf3_ref,
               out_ref):
    i = pl.program_id(0)
    cb = cb_ref[...]
    cT = cT_ref[...]

    diffs = []
    r2 = jnp.full((BLK, N), 1e-16, jnp.float32)
    for d in range(3):
        dd = cb[:, d:d + 1] - cT[d:d + 1, :]
        fr = dd * Li_ref[d]
        fr = fr - jnp.round(fr)
        dd = fr * Ld_ref[d]
        diffs.append(dd)
        r2 = r2 + dd * dd
    r = jnp.sqrt(r2)

    u = (r - RCUT_SMTH) / (RCUT - RCUT_SMTH)
    u = jnp.clip(u, 0.0, 1.0)
    sw = u * u * u * (-6.0 * u * u + 15.0 * u - 10.0) + 1.0
    inv_r = 1.0 / jnp.maximum(r, 1e-8)
    sr = jnp.where(r < RCUT, inv_r * sw, 0.0)
    rows = i * BLK + jax.lax.broadcasted_iota(jnp.int32, (BLK, N), 0)
    cols = jax.lax.broadcasted_iota(jnp.int32, (BLK, N), 1)
    sr = jnp.where(rows == cols, 0.0, sr)

    srn = sr / SR_STD
    snc = (sr - SR_MEAN) / SR_STD
    inv_rr = 1.0 / (r + 1e-16)
    sq3 = 3.0 ** 0.5
    R = [srn] + [sq3 * srn * (diffs[d] * inv_rr) for d in range(3)]

    H = jnp.tanh(W1_ref[...] * snc[None, :, :] + b1_ref[...])

    W2p = W2p_ref[...]
    b2p = b2p_ref[...]

    T = [[] for _ in range(4)]
    for g in range(BLK // GRP):
        Hg = H[:, g * GRP:(g + 1) * GRP, :].reshape(WID1 * GRP, N)
        Gg = jnp.tanh(
            jax.lax.dot_general(W2p, Hg, (((1,), (0,)), ((), ())),
                                preferred_element_type=jnp.float32) + b2p)
        Gg3 = Gg.reshape(GRP, WID2, N)
        for x in range(4):
            Rg = R[x][g * GRP:(g + 1) * GRP, None, :]
            T[x].append(jnp.sum(Gg3 * Rg, axis=2) * (1.0 / NNBRS))
    Tm = [jnp.concatenate(T[x], axis=0) for x in range(4)]

    TN = Tm[0] + tb_ref[...]
    feats = []
    for a in range(AXIS):
        fa = TN * TN[:, a:a + 1]
        for d in range(1, 4):
            fa = fa + Tm[d] * Tm[d][:, a:a + 1]
        feats.append(fa)
    feat = jnp.concatenate(feats, axis=1)

    f1 = jnp.tanh(jnp.dot(feat, Wf1_ref[...],
                          preferred_element_type=jnp.float32) + bf1_ref[...])
    f2 = jnp.tanh(jnp.dot(f1, Wf2_ref[...],
                          preferred_element_type=jnp.float32) + bf2_ref[...])
    v = jnp.dot(jnp.sum(f2, axis=0, keepdims=True), Wf3_ref[...],
                preferred_element_type=jnp.float32)

    @pl.when(i == 0)
    def _init():
        out_ref[...] = jnp.zeros_like(out_ref)

    out_ref[...] += v


def kernel(coord_N3, box_33, W_e1, b_e1, W_e2, b_e2, Tbias,
           W_f1, b_f1, W_f2, b_f2, W_f3, b_f3):
    coordT = coord_N3.T
    Ld = jnp.diagonal(box_33)
    Li = 1.0 / Ld
    W1c = W_e1.reshape(WID1, 1, 1)
    b1c = b_e1.reshape(WID1, 1, 1)
    eye8 = jnp.eye(GRP, dtype=jnp.float32)
    W2p = jnp.einsum('kw,jJ->jwkJ', W_e2, eye8).reshape(GRP * WID2, WID1 * GRP)
    b2p = jnp.tile(b_e2, GRP).reshape(GRP * WID2, 1)
    tb = Tbias.reshape(1, WID2)
    bf1 = b_f1.reshape(1, FIT)
    bf2 = b_f2.reshape(1, FIT)

    grid = (N // BLK,)
    res = pl.pallas_call(
        _dp_kernel,
        grid=grid,
        in_specs=[
            pl.BlockSpec(memory_space=pltpu.SMEM),
            pl.BlockSpec(memory_space=pltpu.SMEM),
            pl.BlockSpec((BLK, 3), lambda i: (i, 0)),
            pl.BlockSpec((3, N), lambda i: (0, 0)),
            pl.BlockSpec((WID1, 1, 1), lambda i: (0, 0, 0)),
            pl.BlockSpec((WID1, 1, 1), lambda i: (0, 0, 0)),
            pl.BlockSpec((GRP * WID2, WID1 * GRP), lambda i: (0, 0)),
            pl.BlockSpec((GRP * WID2, 1), lambda i: (0, 0)),
            pl.BlockSpec((1, WID2), lambda i: (0, 0)),
            pl.BlockSpec((AXIS * WID2, FIT), lambda i: (0, 0)),
            pl.BlockSpec((1, FIT), lambda i: (0, 0)),
            pl.BlockSpec((FIT, FIT), lambda i: (0, 0)),
            pl.BlockSpec((1, FIT), lambda i: (0, 0)),
            pl.BlockSpec((FIT, 1), lambda i: (0, 0)),
        ],
        out_specs=pl.BlockSpec((1, 1), lambda i: (0, 0)),
        out_shape=jax.ShapeDtypeStruct((1, 1), jnp.float32),
    )(Ld, Li, coord_N3, coordT, W1c, b1c, W2p, b2p, tb,
      W_f1, bf1, W_f2, bf2, W_f3)

    return (res[0, 0] + N * (b_f3[0] + EBIAS)) * OUT_NORM
